# baseline (speedup 1.0000x reference)
import functools

import jax
import jax.numpy as jnp
from jax import lax
from jax.experimental import pallas as pl
from jax.experimental.pallas import tpu as pltpu
from jax.experimental.pallas import tpu_sc as plsc

N = 10000
E = 320000
D = 128
DH = 64
H = 8
HH = 4
HD = 16
DE = 4

NC = 2
NS = 16
EB = 128
SPW = 159
NBLK = SPW * NS
E_PAD = NBLK * EB
OUTER = SPW // 3
NPAD = 10240
ROWS_PER_TILE = NPAD // NS


def _node_tables_body(x_ref, wall_ref, asrc_ref, adst_ref,
                      hlo_ref, hhi_ref, as_ref, ad_ref):
    h = jnp.dot(x_ref[...], wall_ref[...], preferred_element_type=jnp.float32)
    hlo_ref[...] = h[:, :DH]
    hhi_ref[...] = h[:, DH:]
    as_ref[...] = jnp.dot(h, asrc_ref[...], preferred_element_type=jnp.float32)
    ad_ref[...] = jnp.dot(h, adst_ref[...], preferred_element_type=jnp.float32)


def _node_tables(x, w_all, a_src16, a_dst16):
    bn = 1024
    return pl.pallas_call(
        _node_tables_body,
        grid=(NPAD // bn,),
        in_specs=[
            pl.BlockSpec((bn, D), lambda i: (i, 0)),
            pl.BlockSpec((D, D), lambda i: (0, 0)),
            pl.BlockSpec((D, 16), lambda i: (0, 0)),
            pl.BlockSpec((D, 16), lambda i: (0, 0)),
        ],
        out_specs=[
            pl.BlockSpec((bn, DH), lambda i: (i, 0)),
            pl.BlockSpec((bn, DH), lambda i: (i, 0)),
            pl.BlockSpec((bn, 16), lambda i: (i, 0)),
            pl.BlockSpec((bn, 16), lambda i: (i, 0)),
        ],
        out_shape=[
            jax.ShapeDtypeStruct((NPAD, DH), jnp.float32),
            jax.ShapeDtypeStruct((NPAD, DH), jnp.float32),
            jax.ShapeDtypeStruct((NPAD, 16), jnp.float32),
            jax.ShapeDtypeStruct((NPAD, 16), jnp.float32),
        ],
    )(x, w_all, a_src16, a_dst16)


def _edge_table_body(ea_ref, wc_ref, ae_ref):
    ae_ref[...] = jnp.dot(ea_ref[...], wc_ref[...], preferred_element_type=jnp.float32)


def _edge_table(edge_attr, wc16):
    be = 2048
    return pl.pallas_call(
        _edge_table_body,
        grid=(E_PAD // be,),
        in_specs=[
            pl.BlockSpec((be, DE), lambda i: (i, 0)),
            pl.BlockSpec((DE, 16), lambda i: (0, 0)),
        ],
        out_specs=pl.BlockSpec((be, 16), lambda i: (i, 0)),
        out_shape=jax.ShapeDtypeStruct((E_PAD, 16), jnp.float32),
    )(edge_attr, wc16)


def _edge_kernel_body(src2_hbm, dst2_hbm, hlo_hbm, hhi_hbm, as_hbm, ad_hbm,
                      ae_hbm, znum_hbm, zden_hbm,
                      numlo_hbm, numhi_hbm, den_hbm,
                      src_i0, src_i1, src_i2, dst_i0, dst_i1, dst_i2,
                      dsts_b0, dsts_b1, dsts_b2,
                      as_b0, as_b1, as_b2, ad_b0, ad_b1, ad_b2,
                      ae_b0, ae_b1, ae_b2, h_b0, h_b1, h_b2,
                      w_b0, w_b1, w_b2,
                      num_sh, den_sh,
                      semg0, semg1, semg2, sems0, sems1, sems2,
                      semi0, semi1, semi2):
    c = lax.axis_index("c")
    sid = lax.axis_index("s")
    src_i = (src_i0, src_i1, src_i2)
    dst_i = (dst_i0, dst_i1, dst_i2)
    dsts_b = (dsts_b0, dsts_b1, dsts_b2)
    as_b = (as_b0, as_b1, as_b2)
    ad_b = (ad_b0, ad_b1, ad_b2)
    ae_b = (ae_b0, ae_b1, ae_b2)
    h_b = (h_b0, h_b1, h_b2)
    w_b = (w_b0, w_b1, w_b2)
    semg = (semg0, semg1, semg2)
    sems = (sems0, sems1, sems2)
    semi = (semi0, semi1, semi2)

    r0 = sid * ROWS_PER_TILE
    pltpu.sync_copy(znum_hbm.at[pl.ds(r0, ROWS_PER_TILE)],
                    num_sh.at[pl.ds(r0, ROWS_PER_TILE)])
    pltpu.sync_copy(zden_hbm.at[pl.ds(r0, ROWS_PER_TILE)],
                    den_sh.at[pl.ds(r0, ROWS_PER_TILE)])
    plsc.subcore_barrier()

    blk0 = sid * SPW
    lanes = lax.iota(jnp.int32, 16)
    headmask = lanes < H

    def idx_descs(t, sl):
        return (
            pltpu.make_async_copy(src2_hbm.at[blk0 + t], src_i[sl], semi[sl]),
            pltpu.make_async_copy(dst2_hbm.at[blk0 + t], dst_i[sl], semi[sl]),
        )

    def run(h_tbl, koff, do_den):
        def gather_descs(t, sl):
            return (
                pltpu.make_async_copy(ae_hbm.at[blk0 + t], ae_b[sl], semg[sl]),
                pltpu.make_async_copy(as_hbm.at[src_i[sl]], as_b[sl], semg[sl]),
                pltpu.make_async_copy(ad_hbm.at[dst_i[sl]], ad_b[sl], semg[sl]),
                pltpu.make_async_copy(h_tbl.at[src_i[sl]], h_b[sl], semg[sl]),
            )

        def start_scatters(t, sl):
            pltpu.async_copy(h_b[sl], num_sh.at[dsts_b[sl]], sems[sl], add=True)
            if do_den:
                pltpu.async_copy(w_b[sl], den_sh.at[dsts_b[sl]], sems[sl],
                                 add=True)

        def wait_scatters(t, sl):
            pltpu.make_async_copy(h_b[sl], num_sh.at[dsts_b[sl]],
                                  sems[sl]).wait()
            if do_den:
                pltpu.make_async_copy(w_b[sl], den_sh.at[dsts_b[sl]],
                                      sems[sl]).wait()

        for tt in (0, 1, 2):
            for d in idx_descs(tt, tt):
                d.start()
        for d in idx_descs(0, 0):
            d.wait()
        for d in idx_descs(1, 1):
            d.wait()
        for d in gather_descs(0, 0):
            d.start()
        for d in gather_descs(1, 1):
            d.start()

        def outer(o, carry):
            for ph in range(3):
                t = 3 * o + ph
                sl = ph
                for d in gather_descs(t, sl):
                    d.wait()

                for i in range(EB // 16):
                    dsts_b[sl][pl.ds(i * 16, 16)] = dst_i[sl][pl.ds(i * 16, 16)]

                @pl.when(t + 3 < SPW)
                def _():
                    for d in idx_descs(t + 3, sl):
                        d.start()

                @plsc.parallel_loop(0, EB, unroll=2)
                def _(j):
                    u = as_b[sl][j, :] + ad_b[sl][j, :] + ae_b[sl][j, :]
                    u = jnp.where(u >= 0.0, u, 0.2 * u)
                    w = jnp.exp(u)
                    w = jnp.where(headmask, w, 0.0)
                    w_b[sl][j, :] = w
                    for k in range(HH):
                        h_b[sl][j, pl.ds(k * HD, HD)] = (
                            h_b[sl][j, pl.ds(k * HD, HD)] * w[koff + k])

                @pl.when(t > 0)
                def _():
                    wait_scatters(t - 1, (sl - 1) % 3)

                @pl.when(t + 2 < SPW)
                def _():
                    for d in idx_descs(t + 2, (sl + 2) % 3):
                        d.wait()
                    for d in gather_descs(t + 2, (sl + 2) % 3):
                        d.start()

                start_scatters(t, sl)
            return carry

        lax.fori_loop(0, OUTER, outer, 0)
        wait_scatters(SPW - 1, 2)

    @pl.when(c == 0)
    def _():
        run(hlo_hbm, 0, True)
        plsc.subcore_barrier()
        pltpu.sync_copy(num_sh.at[pl.ds(r0, ROWS_PER_TILE)],
                        numlo_hbm.at[pl.ds(r0, ROWS_PER_TILE)])
        pltpu.sync_copy(den_sh.at[pl.ds(r0, ROWS_PER_TILE)],
                        den_hbm.at[pl.ds(r0, ROWS_PER_TILE)])

    @pl.when(c == 1)
    def _():
        run(hhi_hbm, HH, False)
        plsc.subcore_barrier()
        pltpu.sync_copy(num_sh.at[pl.ds(r0, ROWS_PER_TILE)],
                        numhi_hbm.at[pl.ds(r0, ROWS_PER_TILE)])


def _edge_phase(src2d, dst2d, h_lo, h_hi, as16, ad16, ae3d):
    znum = jnp.zeros((NPAD, DH), jnp.float32)
    zden = jnp.zeros((NPAD, 16), jnp.float32)
    run = functools.partial(
        pl.kernel,
        out_type=[
            jax.ShapeDtypeStruct((NPAD, DH), jnp.float32),
            jax.ShapeDtypeStruct((NPAD, DH), jnp.float32),
            jax.ShapeDtypeStruct((NPAD, 16), jnp.float32),
        ],
        mesh=plsc.VectorSubcoreMesh(core_axis_name="c", subcore_axis_name="s"),
        compiler_params=pltpu.CompilerParams(use_tc_tiling_on_sc=False),
        scratch_types=(
            [pltpu.VMEM((EB,), jnp.int32)] * 6
            + [pltpu.VMEM((EB,), jnp.int32)] * 3
            + [pltpu.VMEM((EB, 16), jnp.float32)] * 3
            + [pltpu.VMEM((EB, 16), jnp.float32)] * 3
            + [pltpu.VMEM((EB, 16), jnp.float32)] * 3
            + [pltpu.VMEM((EB, DH), jnp.float32)] * 3
            + [pltpu.VMEM((EB, 16), jnp.float32)] * 3
            + [pltpu.VMEM_SHARED((NPAD, DH), jnp.float32),
               pltpu.VMEM_SHARED((NPAD, 16), jnp.float32)]
            + [pltpu.SemaphoreType.DMA] * 9
        ),
    )(_edge_kernel_body)
    return run(src2d, dst2d, h_lo, h_hi, as16, ad16, ae3d, znum, zden)


def _leaky(v, s):
    return jnp.where(v >= 0.0, v, s * v)


def _epilogue_body(numlo_ref, numhi_ref, den_ref, rep_ref,
                   bconv_ref, wfc_ref, bfc_ref, lng_ref, lnb_ref,
                   wgate_ref, bgate_ref, wglob_ref, bglob_ref, out_ref):
    den = den_ref[pl.ds(0, N), :]
    den_rep = jnp.dot(den, rep_ref[...], preferred_element_type=jnp.float32)
    num = jnp.concatenate(
        [numlo_ref[pl.ds(0, N), :], numhi_ref[pl.ds(0, N), :]], axis=-1)
    x = num / (den_rep + 1e-16) + bconv_ref[...]
    t = jnp.dot(x, wfc_ref[...], preferred_element_type=jnp.float32) + bfc_ref[...]
    t = _leaky(t, 0.01)
    t = t - jnp.max(t, axis=-1, keepdims=True)
    et = jnp.exp(t)
    sa = et / jnp.sum(et, axis=-1, keepdims=True)
    x = _leaky(x * sa, 0.2)
    x = jnp.dot(x, wfc_ref[...], preferred_element_type=jnp.float32) + bfc_ref[...]
    mu = jnp.mean(x, axis=-1, keepdims=True)
    xc = x - mu
    var = jnp.mean(xc * xc, axis=-1, keepdims=True)
    x = xc * jax.lax.rsqrt(var + 1e-5) * lng_ref[...] + lnb_ref[...]
    nrm = jnp.sqrt(jnp.sum(x * x, axis=-1, keepdims=True))
    x = x / jnp.maximum(nrm, 1e-12)
    g = jnp.sum(x * wgate_ref[...], axis=-1, keepdims=True) + bgate_ref[0, 0]
    g = g - jnp.max(g)
    eg = jnp.exp(g)
    gate = eg / jnp.sum(eg)
    xg = jnp.sum(gate * x, axis=0, keepdims=True)
    q = jnp.dot(xg, wglob_ref[...], preferred_element_type=jnp.float32) + bglob_ref[...]
    q = jnp.maximum(q, 0.0)
    q = q - jnp.max(q, axis=-1, keepdims=True)
    eq = jnp.exp(q)
    ga = eq / jnp.sum(eq, axis=-1, keepdims=True)
    out_ref[...] = x * ga


def _epilogue(numlo, numhi, den, rep, bconv_row, wfc, bfc_row, lng_row,
              lnb_row, wgate_row, bgate2, wglob, bglob_row):
    return pl.pallas_call(
        _epilogue_body,
        out_shape=jax.ShapeDtypeStruct((N, D), jnp.float32),
    )(numlo, numhi, den, rep, bconv_row, wfc, bfc_row, lng_row,
      lnb_row, wgate_row, bgate2, wglob, bglob_row)


def kernel(x, edge_index, edge_attr, W, att_src, att_dst, W_edge, att_edge,
           b_conv, W_fc, b_fc, ln_g, ln_b, W_gate, b_gate, W_glob, b_glob):
    src = edge_index[0].astype(jnp.int32)
    dst = edge_index[1].astype(jnp.int32)

    w_all = jnp.transpose(W, (1, 0, 2)).reshape(D, H * HD)
    eye = jnp.eye(H, dtype=jnp.float32)
    a_src16 = jnp.concatenate(
        [(eye[:, None, :] * att_src[:, :, None]).reshape(H * HD, H),
         jnp.zeros((H * HD, 16 - H), jnp.float32)], axis=1)
    a_dst16 = jnp.concatenate(
        [(eye[:, None, :] * att_dst[:, :, None]).reshape(H * HD, H),
         jnp.zeros((H * HD, 16 - H), jnp.float32)], axis=1)
    wc16 = jnp.concatenate(
        [jnp.einsum("hdk,hk->dh", W_edge, att_edge),
         jnp.zeros((DE, 16 - H), jnp.float32)], axis=1)
    rep = jnp.concatenate(
        [jnp.repeat(jnp.eye(H, dtype=jnp.float32), HD, axis=1),
         jnp.zeros((16 - H, D), jnp.float32)], axis=0)

    xp = jnp.concatenate([x, jnp.zeros((NPAD - N, D), jnp.float32)], axis=0)
    h_lo, h_hi, as16, ad16 = _node_tables(xp, w_all, a_src16, a_dst16)

    npad_e = E_PAD - E
    src2d = jnp.concatenate([src, jnp.zeros((npad_e,), jnp.int32)]).reshape(
        NBLK, EB)
    dst2d = jnp.concatenate([dst, jnp.full((npad_e,), N, jnp.int32)]).reshape(
        NBLK, EB)
    ea_pad = jnp.concatenate(
        [edge_attr, jnp.zeros((npad_e, DE), jnp.float32)], axis=0)
    ae3d = _edge_table(ea_pad, wc16).reshape(NBLK, EB, 16)

    numlo, numhi, den = _edge_phase(src2d, dst2d, h_lo, h_hi, as16, ad16, ae3d)

    return _epilogue(
        numlo, numhi, den, rep,
        b_conv.reshape(1, D), W_fc, b_fc.reshape(1, D), ln_g.reshape(1, D),
        ln_b.reshape(1, D), W_gate.reshape(1, D), b_gate.reshape(1, 1),
        W_glob, b_glob.reshape(1, D))


# speedup vs baseline: 1.3882x; 1.3882x over previous
import functools

import jax
import jax.numpy as jnp
from jax import lax
from jax.experimental import pallas as pl
from jax.experimental.pallas import tpu as pltpu
from jax.experimental.pallas import tpu_sc as plsc

N = 10000
E = 320000
D = 128
DH = 64
H = 8
HH = 4
HD = 16
DE = 4

NC = 2
NS = 16
EB = 128
SPW = 159
NBLK = SPW * NS
E_PAD = NBLK * EB
EF = E_PAD // 32
OUTER = SPW // 3
NPAD = 10240
ROWS_PER_TILE = NPAD // NS


def _node_tables_body(x_ref, wall_ref, asrc_ref, adst_ref,
                      hlo_ref, hhi_ref, as_ref, ad_ref):
    h = jnp.dot(x_ref[...], wall_ref[...], preferred_element_type=jnp.float32)
    hlo_ref[...] = h[:, :DH]
    hhi_ref[...] = h[:, DH:]
    as_ref[...] = jnp.dot(h, asrc_ref[...], preferred_element_type=jnp.float32)
    ad_ref[...] = jnp.dot(h, adst_ref[...], preferred_element_type=jnp.float32)


def _node_tables(x, w_all, a_src16, a_dst16):
    bn = 1024
    return pl.pallas_call(
        _node_tables_body,
        grid=(NPAD // bn,),
        in_specs=[
            pl.BlockSpec((bn, D), lambda i: (i, 0)),
            pl.BlockSpec((D, D), lambda i: (0, 0)),
            pl.BlockSpec((D, 16), lambda i: (0, 0)),
            pl.BlockSpec((D, 16), lambda i: (0, 0)),
        ],
        out_specs=[
            pl.BlockSpec((bn, DH), lambda i: (i, 0)),
            pl.BlockSpec((bn, DH), lambda i: (i, 0)),
            pl.BlockSpec((bn, 16), lambda i: (i, 0)),
            pl.BlockSpec((bn, 16), lambda i: (i, 0)),
        ],
        out_shape=[
            jax.ShapeDtypeStruct((NPAD, DH), jnp.float32),
            jax.ShapeDtypeStruct((NPAD, DH), jnp.float32),
            jax.ShapeDtypeStruct((NPAD, 16), jnp.float32),
            jax.ShapeDtypeStruct((NPAD, 16), jnp.float32),
        ],
    )(x, w_all, a_src16, a_dst16)


def _edge_table_body(ea_ref, m_ref, ae_ref):
    ae_ref[...] = jnp.dot(ea_ref[...], m_ref[...], preferred_element_type=jnp.float32)


def _edge_table(ea_flat, m_base):
    bn = 1272
    return pl.pallas_call(
        _edge_table_body,
        grid=(EF // bn,),
        in_specs=[
            pl.BlockSpec((bn, 128), lambda i: (i, 0)),
            pl.BlockSpec((128, 512), lambda i: (0, 0)),
        ],
        out_specs=pl.BlockSpec((bn, 512), lambda i: (i, 0)),
        out_shape=jax.ShapeDtypeStruct((EF, 512), jnp.float32),
    )(ea_flat, m_base)


def _edge_kernel_body(src2_hbm, dst2_hbm, hlo_hbm, hhi_hbm, as_hbm, ad_hbm,
                      ae_hbm, znum_hbm, zden_hbm,
                      numlo_hbm, numhi_hbm, den_hbm,
                      src_i0, src_i1, src_i2, dst_i0, dst_i1, dst_i2,
                      dsts_b0, dsts_b1, dsts_b2,
                      as_b0, as_b1, as_b2, ad_b0, ad_b1, ad_b2,
                      ae_b0, ae_b1, ae_b2, h_b0, h_b1, h_b2,
                      w_b0, w_b1, w_b2,
                      num_sh, den_sh,
                      semg0, semg1, semg2, sems0, sems1, sems2,
                      semi0, semi1, semi2):
    c = lax.axis_index("c")
    sid = lax.axis_index("s")
    src_i = (src_i0, src_i1, src_i2)
    dst_i = (dst_i0, dst_i1, dst_i2)
    dsts_b = (dsts_b0, dsts_b1, dsts_b2)
    as_b = (as_b0, as_b1, as_b2)
    ad_b = (ad_b0, ad_b1, ad_b2)
    ae_b = (ae_b0, ae_b1, ae_b2)
    h_b = (h_b0, h_b1, h_b2)
    w_b = (w_b0, w_b1, w_b2)
    semg = (semg0, semg1, semg2)
    sems = (sems0, sems1, sems2)
    semi = (semi0, semi1, semi2)

    r0 = sid * ROWS_PER_TILE
    pltpu.sync_copy(znum_hbm.at[pl.ds(r0, ROWS_PER_TILE)],
                    num_sh.at[pl.ds(r0, ROWS_PER_TILE)])
    pltpu.sync_copy(zden_hbm.at[pl.ds(r0, ROWS_PER_TILE)],
                    den_sh.at[pl.ds(r0, ROWS_PER_TILE)])
    plsc.subcore_barrier()

    blk0 = sid * SPW
    lanes = lax.iota(jnp.int32, 16)
    headmask = lanes < H

    def idx_descs(t, sl):
        return (
            pltpu.make_async_copy(src2_hbm.at[blk0 + t], src_i[sl], semi[sl]),
            pltpu.make_async_copy(dst2_hbm.at[blk0 + t], dst_i[sl], semi[sl]),
        )

    def run(h_tbl, koff, do_den):
        def gather_descs(t, sl):
            return (
                pltpu.make_async_copy(ae_hbm.at[pl.ds(4 * (blk0 + t), 4)],
                                      ae_b[sl], semg[sl]),
                pltpu.make_async_copy(as_hbm.at[src_i[sl]], as_b[sl], semg[sl]),
                pltpu.make_async_copy(ad_hbm.at[dst_i[sl]], ad_b[sl], semg[sl]),
                pltpu.make_async_copy(h_tbl.at[src_i[sl]], h_b[sl], semg[sl]),
            )

        def start_scatters(t, sl):
            pltpu.async_copy(h_b[sl], num_sh.at[dsts_b[sl]], sems[sl], add=True)
            if do_den:
                pltpu.async_copy(w_b[sl], den_sh.at[dsts_b[sl]], sems[sl],
                                 add=True)

        def wait_scatters(t, sl):
            pltpu.make_async_copy(h_b[sl], num_sh.at[dsts_b[sl]],
                                  sems[sl]).wait()
            if do_den:
                pltpu.make_async_copy(w_b[sl], den_sh.at[dsts_b[sl]],
                                      sems[sl]).wait()

        for tt in (0, 1, 2):
            for d in idx_descs(tt, tt):
                d.start()
        for d in idx_descs(0, 0):
            d.wait()
        for d in idx_descs(1, 1):
            d.wait()
        for d in gather_descs(0, 0):
            d.start()
        for d in gather_descs(1, 1):
            d.start()

        def outer(o, carry):
            for ph in range(3):
                t = 3 * o + ph
                sl = ph
                for d in gather_descs(t, sl):
                    d.wait()

                for i in range(EB // 16):
                    dsts_b[sl][pl.ds(i * 16, 16)] = dst_i[sl][pl.ds(i * 16, 16)]

                @pl.when(t + 3 < SPW)
                def _():
                    for d in idx_descs(t + 3, sl):
                        d.start()

                @plsc.parallel_loop(0, EB, unroll=2)
                def _(j):
                    u = (as_b[sl][j, :] + ad_b[sl][j, :]
                         + ae_b[sl][j >> 5, pl.ds(16 * (j & 31), 16)])
                    u = jnp.where(u >= 0.0, u, 0.2 * u)
                    w = jnp.exp(u)
                    w = jnp.where(headmask, w, 0.0)
                    w_b[sl][j, :] = w
                    for k in range(HH):
                        h_b[sl][j, pl.ds(k * HD, HD)] = (
                            h_b[sl][j, pl.ds(k * HD, HD)] * w[koff + k])

                @pl.when(t > 0)
                def _():
                    wait_scatters(t - 1, (sl - 1) % 3)

                @pl.when(t + 2 < SPW)
                def _():
                    for d in idx_descs(t + 2, (sl + 2) % 3):
                        d.wait()
                    for d in gather_descs(t + 2, (sl + 2) % 3):
                        d.start()

                start_scatters(t, sl)
            return carry

        lax.fori_loop(0, OUTER, outer, 0)
        wait_scatters(SPW - 1, 2)

    @pl.when(c == 0)
    def _():
        run(hlo_hbm, 0, True)
        plsc.subcore_barrier()
        pltpu.sync_copy(num_sh.at[pl.ds(r0, ROWS_PER_TILE)],
                        numlo_hbm.at[pl.ds(r0, ROWS_PER_TILE)])
        pltpu.sync_copy(den_sh.at[pl.ds(r0, ROWS_PER_TILE)],
                        den_hbm.at[pl.ds(r0, ROWS_PER_TILE)])

    @pl.when(c == 1)
    def _():
        run(hhi_hbm, HH, False)
        plsc.subcore_barrier()
        pltpu.sync_copy(num_sh.at[pl.ds(r0, ROWS_PER_TILE)],
                        numhi_hbm.at[pl.ds(r0, ROWS_PER_TILE)])


def _edge_phase(src2d, dst2d, h_lo, h_hi, as16, ad16, ae3d):
    znum = jnp.zeros((NPAD, DH), jnp.float32)
    zden = jnp.zeros((NPAD, 16), jnp.float32)
    run = functools.partial(
        pl.kernel,
        out_type=[
            jax.ShapeDtypeStruct((NPAD, DH), jnp.float32),
            jax.ShapeDtypeStruct((NPAD, DH), jnp.float32),
            jax.ShapeDtypeStruct((NPAD, 16), jnp.float32),
        ],
        mesh=plsc.VectorSubcoreMesh(core_axis_name="c", subcore_axis_name="s"),
        compiler_params=pltpu.CompilerParams(use_tc_tiling_on_sc=False),
        scratch_types=(
            [pltpu.VMEM((EB,), jnp.int32)] * 6
            + [pltpu.VMEM((EB,), jnp.int32)] * 3
            + [pltpu.VMEM((EB, 16), jnp.float32)] * 3
            + [pltpu.VMEM((EB, 16), jnp.float32)] * 3
            + [pltpu.VMEM((4, 512), jnp.float32)] * 3
            + [pltpu.VMEM((EB, DH), jnp.float32)] * 3
            + [pltpu.VMEM((EB, 16), jnp.float32)] * 3
            + [pltpu.VMEM_SHARED((NPAD, DH), jnp.float32),
               pltpu.VMEM_SHARED((NPAD, 16), jnp.float32)]
            + [pltpu.SemaphoreType.DMA] * 9
        ),
    )(_edge_kernel_body)
    return run(src2d, dst2d, h_lo, h_hi, as16, ad16, ae3d, znum, zden)


def _leaky(v, s):
    return jnp.where(v >= 0.0, v, s * v)


def _epilogue_body(numlo_ref, numhi_ref, den_ref, rep_ref,
                   bconv_ref, wfc_ref, bfc_ref, lng_ref, lnb_ref,
                   wgate_ref, bgate_ref, wglob_ref, bglob_ref, out_ref):
    den = den_ref[pl.ds(0, N), :]
    den_rep = jnp.dot(den, rep_ref[...], preferred_element_type=jnp.float32)
    num = jnp.concatenate(
        [numlo_ref[pl.ds(0, N), :], numhi_ref[pl.ds(0, N), :]], axis=-1)
    x = num / (den_rep + 1e-16) + bconv_ref[...]
    t = jnp.dot(x, wfc_ref[...], preferred_element_type=jnp.float32) + bfc_ref[...]
    t = _leaky(t, 0.01)
    t = t - jnp.max(t, axis=-1, keepdims=True)
    et = jnp.exp(t)
    sa = et / jnp.sum(et, axis=-1, keepdims=True)
    x = _leaky(x * sa, 0.2)
    x = jnp.dot(x, wfc_ref[...], preferred_element_type=jnp.float32) + bfc_ref[...]
    mu = jnp.mean(x, axis=-1, keepdims=True)
    xc = x - mu
    var = jnp.mean(xc * xc, axis=-1, keepdims=True)
    x = xc * jax.lax.rsqrt(var + 1e-5) * lng_ref[...] + lnb_ref[...]
    nrm = jnp.sqrt(jnp.sum(x * x, axis=-1, keepdims=True))
    x = x / jnp.maximum(nrm, 1e-12)
    g = jnp.sum(x * wgate_ref[...], axis=-1, keepdims=True) + bgate_ref[0, 0]
    g = g - jnp.max(g)
    eg = jnp.exp(g)
    gate = eg / jnp.sum(eg)
    xg = jnp.sum(gate * x, axis=0, keepdims=True)
    q = jnp.dot(xg, wglob_ref[...], preferred_element_type=jnp.float32) + bglob_ref[...]
    q = jnp.maximum(q, 0.0)
    q = q - jnp.max(q, axis=-1, keepdims=True)
    eq = jnp.exp(q)
    ga = eq / jnp.sum(eq, axis=-1, keepdims=True)
    out_ref[...] = x * ga


def _epilogue(numlo, numhi, den, rep, bconv_row, wfc, bfc_row, lng_row,
              lnb_row, wgate_row, bgate2, wglob, bglob_row):
    return pl.pallas_call(
        _epilogue_body,
        out_shape=jax.ShapeDtypeStruct((N, D), jnp.float32),
    )(numlo, numhi, den, rep, bconv_row, wfc, bfc_row, lng_row,
      lnb_row, wgate_row, bgate2, wglob, bglob_row)


def kernel(x, edge_index, edge_attr, W, att_src, att_dst, W_edge, att_edge,
           b_conv, W_fc, b_fc, ln_g, ln_b, W_gate, b_gate, W_glob, b_glob):
    src = edge_index[0].astype(jnp.int32)
    dst = edge_index[1].astype(jnp.int32)

    w_all = jnp.transpose(W, (1, 0, 2)).reshape(D, H * HD)
    eye = jnp.eye(H, dtype=jnp.float32)
    a_src16 = jnp.concatenate(
        [(eye[:, None, :] * att_src[:, :, None]).reshape(H * HD, H),
         jnp.zeros((H * HD, 16 - H), jnp.float32)], axis=1)
    a_dst16 = jnp.concatenate(
        [(eye[:, None, :] * att_dst[:, :, None]).reshape(H * HD, H),
         jnp.zeros((H * HD, 16 - H), jnp.float32)], axis=1)
    wc = jnp.einsum("hdk,hk->dh", W_edge, att_edge)
    wc_pad = jnp.concatenate([wc, jnp.zeros((DE, 16 - H), jnp.float32)], axis=1)
    m_base = jnp.einsum("wv,kh->wkvh", jnp.eye(32, dtype=jnp.float32),
                        wc_pad).reshape(128, 512)
    rep = jnp.concatenate(
        [jnp.repeat(jnp.eye(H, dtype=jnp.float32), HD, axis=1),
         jnp.zeros((16 - H, D), jnp.float32)], axis=0)

    xp = jnp.concatenate([x, jnp.zeros((NPAD - N, D), jnp.float32)], axis=0)
    h_lo, h_hi, as16, ad16 = _node_tables(xp, w_all, a_src16, a_dst16)

    npad_e = E_PAD - E
    src2d = jnp.concatenate([src, jnp.zeros((npad_e,), jnp.int32)]).reshape(
        NBLK, EB)
    dst2d = jnp.concatenate([dst, jnp.full((npad_e,), N, jnp.int32)]).reshape(
        NBLK, EB)
    ea_flat = jnp.concatenate(
        [edge_attr.reshape(E // 32, D),
         jnp.zeros((EF - E // 32, D), jnp.float32)], axis=0)
    ae_pk = _edge_table(ea_flat, m_base)

    numlo, numhi, den = _edge_phase(src2d, dst2d, h_lo, h_hi, as16, ad16, ae_pk)

    return _epilogue(
        numlo, numhi, den, rep,
        b_conv.reshape(1, D), W_fc, b_fc.reshape(1, D), ln_g.reshape(1, D),
        ln_b.reshape(1, D), W_gate.reshape(1, D), b_gate.reshape(1, 1),
        W_glob, b_glob.reshape(1, D))
